# Initial kernel scaffold; baseline (speedup 1.0000x reference)
import jax
import jax.numpy as jnp
from jax.experimental import pallas as pl


def kernel(nfeats, efeats, edge_index, W1, b1, W2, b2):
    raise NotImplementedError("write your pallas kernel here")


# speedup vs baseline: 5.0705x; 5.0705x over previous
import functools

import jax
import jax.numpy as jnp
from jax import lax
from jax.experimental import pallas as pl
from jax.experimental.pallas import tpu as pltpu
from jax.experimental.pallas import tpu_sc as plsc

N, E, D_IN, D_E, D_OUT = 10000, 320000, 128, 16, 128

_NC, _NS = 2, 16
_NW = _NC * _NS
_EPW = E // _NW
_CH = 80
_NCH = _EPW // _CH
_RB = 2000
_NRB = _EPW // _RB
_CPB = _RB // _CH
_NPAD = 10240
_RPS = _NPAD // _NS


def _seg_sum_body(efeats_hbm, dst_hbm, out_hbm, idx_v, rows_v, stage_v, acc_sh):
  c = lax.axis_index("c")
  s = lax.axis_index("s")
  wid = s * _NC + c

  zeros16 = jnp.zeros((16,), jnp.float32)

  def zero_body(i, carry):
    stage_v[i, :] = zeros16
    return carry

  lax.fori_loop(0, _RPS, zero_body, 0)
  pltpu.sync_copy(stage_v, acc_sh.at[pl.ds(s * _RPS, _RPS), :])

  pltpu.sync_copy(dst_hbm.at[wid], idx_v)
  plsc.subcore_barrier()

  for b in range(_NRB):
    pltpu.sync_copy(
        efeats_hbm.at[pl.ds(wid * _EPW + b * _RB, _RB), :], rows_v
    )

    def scatter_body(k, carry, b=b):
      pltpu.sync_copy(
          rows_v.at[pl.ds(k * _CH, _CH), :],
          acc_sh.at[idx_v.at[b * _CPB + k]],
          add=True,
      )
      return carry

    lax.fori_loop(0, _CPB, scatter_body, 0)

  plsc.subcore_barrier()

  pltpu.sync_copy(acc_sh.at[pl.ds(s * _RPS, _RPS), :], stage_v)
  pltpu.sync_copy(stage_v, out_hbm.at[c, pl.ds(s * _RPS, _RPS), :])


def _segment_sum_sc(efeats, dst2d):
  mesh = plsc.VectorSubcoreMesh(
      core_axis_name="c", subcore_axis_name="s",
      num_cores=_NC, num_subcores=_NS,
  )
  return pl.kernel(
      _seg_sum_body,
      out_type=jax.ShapeDtypeStruct((_NC, _NPAD, D_E), jnp.float32),
      mesh=mesh,
      scratch_types=[
          pltpu.VMEM((_NCH, _CH), jnp.int32),
          pltpu.VMEM((_RB, D_E), jnp.float32),
          pltpu.VMEM((_RPS, D_E), jnp.float32),
          pltpu.VMEM_SHARED((_NPAD, D_E), jnp.float32),
      ],
      compiler_params=pltpu.CompilerParams(use_tc_tiling_on_sc=False),
  )(efeats, dst2d)


_RBLK = 2000


def _mlp_body(nf_ref, p_ref, w1a_ref, w1b_ref, b1_ref, w2_ref, b2_ref, out_ref):
  hn = p_ref[0] + p_ref[1]
  x = jnp.dot(nf_ref[:], w1a_ref[:], preferred_element_type=jnp.float32)
  x = x + jnp.dot(hn, w1b_ref[:], preferred_element_type=jnp.float32)
  h1 = jnp.maximum(x + b1_ref[:], 0.0)
  y = jnp.dot(h1, w2_ref[:], preferred_element_type=jnp.float32) + b2_ref[:]
  out_ref[:] = jnp.maximum(y, 0.0)


def _mlp_tc(nfeats, partials, w1a, w1b, b1, w2, b2):
  grid = (N // _RBLK,)
  return pl.pallas_call(
      _mlp_body,
      grid=grid,
      in_specs=[
          pl.BlockSpec((_RBLK, D_IN), lambda i: (i, 0)),
          pl.BlockSpec((_NC, _RBLK, D_E), lambda i: (0, i, 0)),
          pl.BlockSpec((D_IN, D_OUT), lambda i: (0, 0)),
          pl.BlockSpec((D_E, D_OUT), lambda i: (0, 0)),
          pl.BlockSpec((1, D_OUT), lambda i: (0, 0)),
          pl.BlockSpec((D_OUT, D_OUT), lambda i: (0, 0)),
          pl.BlockSpec((1, D_OUT), lambda i: (0, 0)),
      ],
      out_specs=pl.BlockSpec((_RBLK, D_OUT), lambda i: (i, 0)),
      out_shape=jax.ShapeDtypeStruct((N, D_OUT), jnp.float32),
  )(nfeats, partials, w1a, w1b, b1, w2, b2)


def kernel(nfeats, efeats, edge_index, W1, b1, W2, b2):
  dst2d = edge_index[1].astype(jnp.int32).reshape(_NW, _NCH, _CH)
  partials = _segment_sum_sc(efeats, dst2d)[:, :N]
  w1a = W1[:D_IN]
  w1b = W1[D_IN:]
  return _mlp_tc(
      nfeats, partials, w1a, w1b,
      b1.reshape(1, D_OUT), W2, b2.reshape(1, D_OUT),
  )


# speedup vs baseline: 5.2632x; 1.0380x over previous
import functools

import jax
import jax.numpy as jnp
from jax import lax
from jax.experimental import pallas as pl
from jax.experimental.pallas import tpu as pltpu
from jax.experimental.pallas import tpu_sc as plsc

N, E, D_IN, D_E, D_OUT = 10000, 320000, 128, 16, 128

_NC, _NS = 2, 16
_NW = _NC * _NS
_EPW = E // _NW
_CH = 80
_NCH = _EPW // _CH
_RB = 2000
_NRB = _EPW // _RB
_CPB = _RB // _CH
_NPAD = 10240
_RPS = _NPAD // _NS


def _seg_sum_body(efeats_hbm, dst_hbm, out_hbm, idx_v, rows_v, stage_v, acc_sh):
  c = lax.axis_index("c")
  s = lax.axis_index("s")
  wid = s * _NC + c

  zeros16 = jnp.zeros((16,), jnp.float32)

  def zero_body(i, carry):
    stage_v[i, :] = zeros16
    return carry

  lax.fori_loop(0, _RPS, zero_body, 0)
  pltpu.sync_copy(stage_v, acc_sh.at[pl.ds(s * _RPS, _RPS), :])

  pltpu.sync_copy(dst_hbm.at[1, wid], idx_v)
  plsc.subcore_barrier()

  for b in range(_NRB):
    pltpu.sync_copy(
        efeats_hbm.at[pl.ds(wid * _EPW + b * _RB, _RB), :], rows_v
    )

    def scatter_body(k, carry, b=b):
      pltpu.sync_copy(
          rows_v.at[pl.ds(k * _CH, _CH), :],
          acc_sh.at[idx_v.at[b * _CPB + k]],
          add=True,
      )
      return carry

    lax.fori_loop(0, _CPB, scatter_body, 0)

  plsc.subcore_barrier()

  pltpu.sync_copy(acc_sh.at[pl.ds(s * _RPS, _RPS), :], stage_v)
  pltpu.sync_copy(stage_v, out_hbm.at[c, pl.ds(s * _RPS, _RPS), :])


def _segment_sum_sc(efeats, dst2d):
  mesh = plsc.VectorSubcoreMesh(
      core_axis_name="c", subcore_axis_name="s",
      num_cores=_NC, num_subcores=_NS,
  )
  return pl.kernel(
      _seg_sum_body,
      out_type=jax.ShapeDtypeStruct((_NC, _NPAD, D_E), jnp.float32),
      mesh=mesh,
      scratch_types=[
          pltpu.VMEM((_NCH, _CH), jnp.int32),
          pltpu.VMEM((_RB, D_E), jnp.float32),
          pltpu.VMEM((_RPS, D_E), jnp.float32),
          pltpu.VMEM_SHARED((_NPAD, D_E), jnp.float32),
      ],
      compiler_params=pltpu.CompilerParams(use_tc_tiling_on_sc=False),
  )(efeats, dst2d)


_RBLK = 2000


def _mlp_body(nf_ref, p_ref, w1a_ref, w1b_ref, b1_ref, w2_ref, b2_ref, out_ref):
  hn = p_ref[0] + p_ref[1]
  x = jnp.dot(nf_ref[:], w1a_ref[:], preferred_element_type=jnp.float32)
  x = x + jnp.dot(hn, w1b_ref[:], preferred_element_type=jnp.float32)
  h1 = jnp.maximum(x + b1_ref[:], 0.0)
  y = jnp.dot(h1, w2_ref[:], preferred_element_type=jnp.float32) + b2_ref[:]
  out_ref[:] = jnp.maximum(y, 0.0)


def _mlp_tc(nfeats, partials, w1a, w1b, b1, w2, b2):
  grid = (N // _RBLK,)
  return pl.pallas_call(
      _mlp_body,
      grid=grid,
      in_specs=[
          pl.BlockSpec((_RBLK, D_IN), lambda i: (i, 0)),
          pl.BlockSpec((_NC, _RBLK, D_E), lambda i: (0, i, 0)),
          pl.BlockSpec((D_IN, D_OUT), lambda i: (0, 0)),
          pl.BlockSpec((D_E, D_OUT), lambda i: (0, 0)),
          pl.BlockSpec((1, D_OUT), lambda i: (0, 0)),
          pl.BlockSpec((D_OUT, D_OUT), lambda i: (0, 0)),
          pl.BlockSpec((1, D_OUT), lambda i: (0, 0)),
      ],
      out_specs=pl.BlockSpec((_RBLK, D_OUT), lambda i: (i, 0)),
      out_shape=jax.ShapeDtypeStruct((N, D_OUT), jnp.float32),
  )(nfeats, partials, w1a, w1b, b1, w2, b2)


def kernel(nfeats, efeats, edge_index, W1, b1, W2, b2):
  ei4d = edge_index.reshape(2, _NW, _NCH, _CH)
  partials = _segment_sum_sc(efeats, ei4d)
  w1a = W1[:D_IN]
  w1b = W1[D_IN:]
  return _mlp_tc(
      nfeats, partials, w1a, w1b,
      b1.reshape(1, D_OUT), W2, b2.reshape(1, D_OUT),
  )


# speedup vs baseline: 5.5494x; 1.0544x over previous
import functools

import jax
import jax.numpy as jnp
from jax import lax
from jax.experimental import pallas as pl
from jax.experimental.pallas import tpu as pltpu
from jax.experimental.pallas import tpu_sc as plsc

N, E, D_IN, D_E, D_OUT = 10000, 320000, 128, 16, 128

_NC, _NS = 2, 16
_NW = _NC * _NS
_EPW = E // _NW
_CH = 80
_NCH = _EPW // _CH
_RB = 2000
_NRB = _EPW // _RB
_CPB = _RB // _CH
_NPAD = 10240
_RPS = _NPAD // _NS


def _seg_sum_body(
    efeats_hbm, dst_hbm, out_hbm,
    idx_v, rows_v0, rows_v1, stage_v, acc_sh, lsem0, lsem1, ssem,
):
  c = lax.axis_index("c")
  s = lax.axis_index("s")
  wid = s * _NC + c
  bufs = (rows_v0, rows_v1)
  lsems = (lsem0, lsem1)

  zeros16 = jnp.zeros((16,), jnp.float32)

  def zero_body(i, carry):
    stage_v[i, :] = zeros16
    return carry

  lax.fori_loop(0, _RPS, zero_body, 0)
  pltpu.sync_copy(stage_v, acc_sh.at[pl.ds(s * _RPS, _RPS), :])

  pltpu.sync_copy(dst_hbm.at[1, wid], idx_v)
  plsc.subcore_barrier()

  def start_load(b):
    return pltpu.async_copy(
        efeats_hbm.at[pl.ds(wid * _EPW + b * _RB, _RB), :],
        bufs[b % 2],
        lsems[b % 2],
    )

  descs = [None] * _NRB
  descs[0] = start_load(0)
  for b in range(_NRB):
    descs[b].wait()
    buf = bufs[b % 2]

    def fire_body(k, carry, buf=buf, b=b):
      pltpu.async_copy(
          buf.at[pl.ds(k * _CH, _CH), :],
          acc_sh.at[idx_v.at[b * _CPB + k]],
          ssem,
          add=True,
      )
      return carry

    lax.fori_loop(0, _CPB, fire_body, 0)
    if b + 1 < _NRB:
      descs[b + 1] = start_load(b + 1)

    def drain_body(k, carry, buf=buf, b=b):
      pltpu.make_async_copy(
          buf.at[pl.ds(k * _CH, _CH), :],
          acc_sh.at[idx_v.at[b * _CPB + k]],
          ssem,
      ).wait()
      return carry

    lax.fori_loop(0, _CPB, drain_body, 0)

  plsc.subcore_barrier()

  pltpu.sync_copy(acc_sh.at[pl.ds(s * _RPS, _RPS), :], stage_v)
  pltpu.sync_copy(stage_v, out_hbm.at[c, pl.ds(s * _RPS, _RPS), :])


def _segment_sum_sc(efeats, dst2d):
  mesh = plsc.VectorSubcoreMesh(
      core_axis_name="c", subcore_axis_name="s",
      num_cores=_NC, num_subcores=_NS,
  )
  return pl.kernel(
      _seg_sum_body,
      out_type=jax.ShapeDtypeStruct((_NC, _NPAD, D_E), jnp.float32),
      mesh=mesh,
      scratch_types=[
          pltpu.VMEM((_NCH, _CH), jnp.int32),
          pltpu.VMEM((_RB, D_E), jnp.float32),
          pltpu.VMEM((_RB, D_E), jnp.float32),
          pltpu.VMEM((_RPS, D_E), jnp.float32),
          pltpu.VMEM_SHARED((_NPAD, D_E), jnp.float32),
          pltpu.SemaphoreType.DMA,
          pltpu.SemaphoreType.DMA,
          pltpu.SemaphoreType.DMA,
      ],
      compiler_params=pltpu.CompilerParams(use_tc_tiling_on_sc=False),
  )(efeats, dst2d)


_RBLK = 2000


def _mlp_body(nf_ref, p_ref, w1a_ref, w1b_ref, b1_ref, w2_ref, b2_ref, out_ref):
  hn = p_ref[0] + p_ref[1]
  x = jnp.dot(nf_ref[:], w1a_ref[:], preferred_element_type=jnp.float32)
  x = x + jnp.dot(hn, w1b_ref[:], preferred_element_type=jnp.float32)
  h1 = jnp.maximum(x + b1_ref[:], 0.0)
  y = jnp.dot(h1, w2_ref[:], preferred_element_type=jnp.float32) + b2_ref[:]
  out_ref[:] = jnp.maximum(y, 0.0)


def _mlp_tc(nfeats, partials, w1a, w1b, b1, w2, b2):
  grid = (N // _RBLK,)
  return pl.pallas_call(
      _mlp_body,
      grid=grid,
      in_specs=[
          pl.BlockSpec((_RBLK, D_IN), lambda i: (i, 0)),
          pl.BlockSpec((_NC, _RBLK, D_E), lambda i: (0, i, 0)),
          pl.BlockSpec((D_IN, D_OUT), lambda i: (0, 0)),
          pl.BlockSpec((D_E, D_OUT), lambda i: (0, 0)),
          pl.BlockSpec((1, D_OUT), lambda i: (0, 0)),
          pl.BlockSpec((D_OUT, D_OUT), lambda i: (0, 0)),
          pl.BlockSpec((1, D_OUT), lambda i: (0, 0)),
      ],
      out_specs=pl.BlockSpec((_RBLK, D_OUT), lambda i: (i, 0)),
      out_shape=jax.ShapeDtypeStruct((N, D_OUT), jnp.float32),
  )(nfeats, partials, w1a, w1b, b1, w2, b2)


def kernel(nfeats, efeats, edge_index, W1, b1, W2, b2):
  ei4d = edge_index.reshape(2, _NW, _NCH, _CH)
  partials = _segment_sum_sc(efeats, ei4d)
  w1a = W1[:D_IN]
  w1b = W1[D_IN:]
  return _mlp_tc(
      nfeats, partials, w1a, w1b,
      b1.reshape(1, D_OUT), W2, b2.reshape(1, D_OUT),
  )


# speedup vs baseline: 6.8880x; 1.2412x over previous
import functools

import jax
import jax.numpy as jnp
from jax import lax
from jax.experimental import pallas as pl
from jax.experimental.pallas import tpu as pltpu
from jax.experimental.pallas import tpu_sc as plsc

N, E, D_IN, D_E, D_OUT = 10000, 320000, 128, 16, 128

_NC, _NS = 2, 16
_NW = _NC * _NS
_EPW = E // _NW
_CH = 80
_NCH = _EPW // _CH
_RB = 2000
_NRB = _EPW // _RB
_CPB = _RB // _CH
_NPAD = 10240
_RPS = _NPAD // _NS


def _seg_sum_body(
    ef_t_hbm, dst_hbm, out_hbm,
    idx_v, tbuf0, tbuf1, rows_v, stage_v, acc_sh, lsem0, lsem1, ssem,
):
  c = lax.axis_index("c")
  s = lax.axis_index("s")
  wid = s * _NC + c
  tbufs = (tbuf0, tbuf1)
  lsems = (lsem0, lsem1)

  zeros16 = jnp.zeros((16,), jnp.float32)

  def zero_body(i, carry):
    stage_v[i, :] = zeros16
    return carry

  lax.fori_loop(0, _RPS, zero_body, 0)
  pltpu.sync_copy(stage_v, acc_sh.at[pl.ds(s * _RPS, _RPS), :])

  pltpu.sync_copy(dst_hbm.at[1, wid], idx_v)
  plsc.subcore_barrier()

  def start_load(b):
    return pltpu.async_copy(
        ef_t_hbm.at[:, pl.ds(wid * _EPW + b * _RB, _RB)],
        tbufs[b % 2],
        lsems[b % 2],
    )

  iota16 = lax.iota(jnp.int32, 16)

  descs = [None] * _NRB
  descs[0] = start_load(0)
  for b in range(_NRB):
    descs[b].wait()
    tbuf = tbufs[b % 2]
    if b + 1 < _NRB:
      descs[b + 1] = start_load(b + 1)

    def transpose_body(k, carry, tbuf=tbuf):
      e0 = k * 8
      for u in range(8):
        col = plsc.load_gather(
            tbuf, [iota16, jnp.full((16,), e0 + u, jnp.int32)]
        )
        rows_v[e0 + u, :] = col
      return carry

    lax.fori_loop(0, _RB // 8, transpose_body, 0)

    def fire_body(k, carry, b=b):
      pltpu.async_copy(
          rows_v.at[pl.ds(k * _CH, _CH), :],
          acc_sh.at[idx_v.at[b * _CPB + k]],
          ssem,
          add=True,
      )
      return carry

    lax.fori_loop(0, _CPB, fire_body, 0)

    def drain_body(k, carry, b=b):
      pltpu.make_async_copy(
          rows_v.at[pl.ds(k * _CH, _CH), :],
          acc_sh.at[idx_v.at[b * _CPB + k]],
          ssem,
      ).wait()
      return carry

    lax.fori_loop(0, _CPB, drain_body, 0)

  plsc.subcore_barrier()

  pltpu.sync_copy(acc_sh.at[pl.ds(s * _RPS, _RPS), :], stage_v)
  pltpu.sync_copy(stage_v, out_hbm.at[c, pl.ds(s * _RPS, _RPS), :])


def _segment_sum_sc(efeats, dst2d):
  mesh = plsc.VectorSubcoreMesh(
      core_axis_name="c", subcore_axis_name="s",
      num_cores=_NC, num_subcores=_NS,
  )
  return pl.kernel(
      _seg_sum_body,
      out_type=jax.ShapeDtypeStruct((_NC, _NPAD, D_E), jnp.float32),
      mesh=mesh,
      scratch_types=[
          pltpu.VMEM((_NCH, _CH), jnp.int32),
          pltpu.VMEM((16, _RB), jnp.float32),
          pltpu.VMEM((16, _RB), jnp.float32),
          pltpu.VMEM((_RB, D_E), jnp.float32),
          pltpu.VMEM((_RPS, D_E), jnp.float32),
          pltpu.VMEM_SHARED((_NPAD, D_E), jnp.float32),
          pltpu.SemaphoreType.DMA,
          pltpu.SemaphoreType.DMA,
          pltpu.SemaphoreType.DMA,
      ],
      compiler_params=pltpu.CompilerParams(
          use_tc_tiling_on_sc=False, needs_layout_passes=False
      ),
  )(efeats, dst2d)


_RBLK = 2000


def _mlp_body(nf_ref, p_ref, w1a_ref, w1b_ref, b1_ref, w2_ref, b2_ref, out_ref):
  hn = p_ref[0] + p_ref[1]
  x = jnp.dot(nf_ref[:], w1a_ref[:], preferred_element_type=jnp.float32)
  x = x + jnp.dot(hn, w1b_ref[:], preferred_element_type=jnp.float32)
  h1 = jnp.maximum(x + b1_ref[:], 0.0)
  y = jnp.dot(h1, w2_ref[:], preferred_element_type=jnp.float32) + b2_ref[:]
  out_ref[:] = jnp.maximum(y, 0.0)


def _mlp_tc(nfeats, partials, w1a, w1b, b1, w2, b2):
  grid = (N // _RBLK,)
  return pl.pallas_call(
      _mlp_body,
      grid=grid,
      in_specs=[
          pl.BlockSpec((_RBLK, D_IN), lambda i: (i, 0)),
          pl.BlockSpec((_NC, _RBLK, D_E), lambda i: (0, i, 0)),
          pl.BlockSpec((D_IN, D_OUT), lambda i: (0, 0)),
          pl.BlockSpec((D_E, D_OUT), lambda i: (0, 0)),
          pl.BlockSpec((1, D_OUT), lambda i: (0, 0)),
          pl.BlockSpec((D_OUT, D_OUT), lambda i: (0, 0)),
          pl.BlockSpec((1, D_OUT), lambda i: (0, 0)),
      ],
      out_specs=pl.BlockSpec((_RBLK, D_OUT), lambda i: (i, 0)),
      out_shape=jax.ShapeDtypeStruct((N, D_OUT), jnp.float32),
  )(nfeats, partials, w1a, w1b, b1, w2, b2)


def kernel(nfeats, efeats, edge_index, W1, b1, W2, b2):
  ei4d = edge_index.reshape(2, _NW, _NCH, _CH)
  partials = _segment_sum_sc(efeats.T, ei4d)
  w1a = W1[:D_IN]
  w1b = W1[D_IN:]
  return _mlp_tc(
      nfeats, partials, w1a, w1b,
      b1.reshape(1, D_OUT), W2, b2.reshape(1, D_OUT),
  )


# speedup vs baseline: 8.1738x; 1.1867x over previous
import functools

import jax
import jax.numpy as jnp
from jax import lax
from jax.experimental import pallas as pl
from jax.experimental.pallas import tpu as pltpu
from jax.experimental.pallas import tpu_sc as plsc

N, E, D_IN, D_E, D_OUT = 10000, 320000, 128, 16, 128

_NC, _NS = 2, 16
_NW = _NC * _NS
_EPW = E // _NW
_CH = 80
_NCH = _EPW // _CH
_RB = 2000
_NRB = _EPW // _RB
_CPB = _RB // _CH
_NPAD = 10240
_RPS = _NPAD // _NS


def _seg_sum_body(
    ef_t_hbm, dst_hbm, out_hbm,
    idx_v, tbuf0, tbuf1, rows_v, stage_v, acc_sh, lsem0, lsem1, ssem,
):
  c = lax.axis_index("c")
  s = lax.axis_index("s")
  wid = s * _NC + c
  tbufs = (tbuf0, tbuf1)
  lsems = (lsem0, lsem1)

  zeros16 = jnp.zeros((16,), jnp.float32)

  def zero_body(i, carry):
    stage_v[i, :] = zeros16
    return carry

  lax.fori_loop(0, _RPS, zero_body, 0)
  pltpu.sync_copy(stage_v, acc_sh.at[pl.ds(s * _RPS, _RPS), :])

  pltpu.sync_copy(dst_hbm.at[1, wid], idx_v)
  plsc.subcore_barrier()

  def start_load(b):
    return pltpu.async_copy(
        ef_t_hbm.at[:, pl.ds(wid * _EPW + b * _RB, _RB)],
        tbufs[b % 2].at[:, pl.ds(0, _RB)],
        lsems[b % 2],
    )

  iota16 = lax.iota(jnp.int32, 16)

  descs = [None] * _NRB
  descs[0] = start_load(0)
  for b in range(_NRB):
    descs[b].wait()
    tbuf = tbufs[b % 2]
    if b + 1 < _NRB:
      descs[b + 1] = start_load(b + 1)

    def transpose_body(k, carry, tbuf=tbuf):
      e0 = k * 8
      for u in range(8):
        col = plsc.load_gather(
            tbuf, [iota16, jnp.full((16,), e0 + u, jnp.int32)]
        )
        rows_v[e0 + u, :] = col
      return carry

    lax.fori_loop(0, _RB // 8, transpose_body, 0)

    def fire_body(k, carry, b=b):
      pltpu.async_copy(
          rows_v.at[pl.ds(k * _CH, _CH), :],
          acc_sh.at[idx_v.at[b * _CPB + k]],
          ssem,
          add=True,
      )
      return carry

    lax.fori_loop(0, _CPB, fire_body, 0)

    def drain_body(k, carry, b=b):
      pltpu.make_async_copy(
          rows_v.at[pl.ds(k * _CH, _CH), :],
          acc_sh.at[idx_v.at[b * _CPB + k]],
          ssem,
      ).wait()
      return carry

    lax.fori_loop(0, _CPB, drain_body, 0)

  plsc.subcore_barrier()

  pltpu.sync_copy(acc_sh.at[pl.ds(s * _RPS, _RPS), :], stage_v)
  pltpu.sync_copy(stage_v, out_hbm.at[c, pl.ds(s * _RPS, _RPS), :])


def _segment_sum_sc(efeats, dst2d):
  mesh = plsc.VectorSubcoreMesh(
      core_axis_name="c", subcore_axis_name="s",
      num_cores=_NC, num_subcores=_NS,
  )
  return pl.kernel(
      _seg_sum_body,
      out_type=jax.ShapeDtypeStruct((_NC, _NPAD, D_E), jnp.float32),
      mesh=mesh,
      scratch_types=[
          pltpu.VMEM((_NCH, _CH), jnp.int32),
          pltpu.VMEM((16, _RB + 1), jnp.float32),
          pltpu.VMEM((16, _RB + 1), jnp.float32),
          pltpu.VMEM((_RB, D_E), jnp.float32),
          pltpu.VMEM((_RPS, D_E), jnp.float32),
          pltpu.VMEM_SHARED((_NPAD, D_E), jnp.float32),
          pltpu.SemaphoreType.DMA,
          pltpu.SemaphoreType.DMA,
          pltpu.SemaphoreType.DMA,
      ],
      compiler_params=pltpu.CompilerParams(
          use_tc_tiling_on_sc=False, needs_layout_passes=False
      ),
  )(efeats, dst2d)


_RBLK = 2000


def _mlp_body(nf_ref, p_ref, w1a_ref, w1b_ref, b1_ref, w2_ref, b2_ref, out_ref):
  hn = p_ref[0] + p_ref[1]
  x = jnp.dot(nf_ref[:], w1a_ref[:], preferred_element_type=jnp.float32)
  x = x + jnp.dot(hn, w1b_ref[:], preferred_element_type=jnp.float32)
  h1 = jnp.maximum(x + b1_ref[:], 0.0)
  y = jnp.dot(h1, w2_ref[:], preferred_element_type=jnp.float32) + b2_ref[:]
  out_ref[:] = jnp.maximum(y, 0.0)


def _mlp_tc(nfeats, partials, w1a, w1b, b1, w2, b2):
  grid = (N // _RBLK,)
  return pl.pallas_call(
      _mlp_body,
      grid=grid,
      in_specs=[
          pl.BlockSpec((_RBLK, D_IN), lambda i: (i, 0)),
          pl.BlockSpec((_NC, _RBLK, D_E), lambda i: (0, i, 0)),
          pl.BlockSpec((D_IN, D_OUT), lambda i: (0, 0)),
          pl.BlockSpec((D_E, D_OUT), lambda i: (0, 0)),
          pl.BlockSpec((1, D_OUT), lambda i: (0, 0)),
          pl.BlockSpec((D_OUT, D_OUT), lambda i: (0, 0)),
          pl.BlockSpec((1, D_OUT), lambda i: (0, 0)),
      ],
      out_specs=pl.BlockSpec((_RBLK, D_OUT), lambda i: (i, 0)),
      out_shape=jax.ShapeDtypeStruct((N, D_OUT), jnp.float32),
  )(nfeats, partials, w1a, w1b, b1, w2, b2)


def kernel(nfeats, efeats, edge_index, W1, b1, W2, b2):
  ei4d = edge_index.reshape(2, _NW, _NCH, _CH)
  partials = _segment_sum_sc(efeats.T, ei4d)
  w1a = W1[:D_IN]
  w1b = W1[D_IN:]
  return _mlp_tc(
      nfeats, partials, w1a, w1b,
      b1.reshape(1, D_OUT), W2, b2.reshape(1, D_OUT),
  )
